# Initial kernel scaffold; baseline (speedup 1.0000x reference)
import jax
import jax.numpy as jnp
from jax.experimental import pallas as pl


def kernel(x, cw0, cb0, cw1, cb1, cw2, cb2, cw3, cb3, cw4, cb4, lw0, lb0, lw1, lb1, lw2, lb2, lw3, lb3):
    raise NotImplementedError("write your pallas kernel here")


# speedup vs baseline: 1.0538x; 1.0538x over previous
import functools

import jax
import jax.numpy as jnp
from jax.experimental import pallas as pl
from jax.experimental.pallas import tpu as pltpu

KS = 5
POOL = 3


def _fill_pstack(pstack_ref, x_ref, Cs, Pf, Mb, base):
    for dy in range(KS):
        for dx in range(KS):
            t = dy * KS + dx
            pstack_ref[pl.ds(t * Cs, Cs), :] = (
                x_ref[0, :, pl.ds(base + dy * Pf + dx, Mb)])


def _pool_rows(o_ref, hm_ref, Cout, Pf, Hp, out_base):
    for hh in range(Hp):
        r = 3 * hh * Pf
        row = jnp.maximum(
            jnp.maximum(hm_ref[:, pl.ds(r, Pf)],
                        hm_ref[:, pl.ds(r + Pf, Pf)]),
            hm_ref[:, pl.ds(r + 2 * Pf, Pf)])
        o_ref[0, :, pl.ds((out_base + hh) * Pf, Pf)] = row


def _conv0_kernel(x_ref, w_ref, b_ref, o_ref, pstack_ref, y_ref, hm_ref):
    Cs, Pf, TH = 8, 214, 105
    Mb = TH * Pf
    Cout = 16
    for band in range(2):
        base = band * Mb
        _fill_pstack(pstack_ref, x_ref, Cs, Pf, Mb, base)
        z = jnp.dot(w_ref[...], pstack_ref[...],
                    preferred_element_type=jnp.float32)
        y_ref[...] = jnp.maximum(z + b_ref[...], 0.0).astype(y_ref.dtype)
        hm_ref[:, pl.ds(0, Mb - 2)] = jnp.maximum(
            jnp.maximum(y_ref[:, pl.ds(0, Mb - 2)],
                        y_ref[:, pl.ds(1, Mb - 2)]),
            y_ref[:, pl.ds(2, Mb - 2)])
        hm_ref[:, pl.ds(Mb - 2, 2)] = jnp.zeros((Cout, 2), hm_ref.dtype)
        _pool_rows(o_ref, hm_ref, Cout, Pf, TH // POOL, band * (TH // POOL))


def _conv12_kernel(x_ref, w1_ref, b1_ref, w2_ref, b2_ref, o_ref,
                   ps1_ref, x2_ref, ps2_ref, y2_ref, hm2_ref):
    Pf = 72
    Mb1 = 68 * Pf
    _fill_pstack(ps1_ref, x_ref, 16, Pf, Mb1, 0)
    z1 = jnp.dot(w1_ref[...], ps1_ref[...],
                 preferred_element_type=jnp.float32)
    y1 = jnp.maximum(z1 + b1_ref[...], 0.0)
    col = jax.lax.rem(jax.lax.broadcasted_iota(jnp.int32, (1, Mb1), 1), Pf)
    y1 = jnp.where(col < 68, y1, 0.0).astype(x2_ref.dtype)
    x2_ref[:, pl.ds(0, Pf + 1)] = jnp.zeros((32, Pf + 1), x2_ref.dtype)
    x2_ref[:, pl.ds(Pf + 1, Mb1)] = y1
    x2_ref[:, pl.ds(Pf + 1 + Mb1, 3 * Pf - 1)] = (
        jnp.zeros((32, 3 * Pf - 1), x2_ref.dtype))
    Mb2 = 66 * Pf
    for dy in range(KS):
        for dx in range(KS):
            t = dy * KS + dx
            ps2_ref[pl.ds(t * 32, 32), :] = (
                x2_ref[:, pl.ds(dy * Pf + dx, Mb2)])
    z2 = jnp.dot(w2_ref[...], ps2_ref[...],
                 preferred_element_type=jnp.float32)
    y2_ref[...] = jnp.maximum(z2 + b2_ref[...], 0.0).astype(y2_ref.dtype)
    hm2_ref[:, pl.ds(0, Mb2 - 2)] = jnp.maximum(
        jnp.maximum(y2_ref[:, pl.ds(0, Mb2 - 2)],
                    y2_ref[:, pl.ds(1, Mb2 - 2)]),
        y2_ref[:, pl.ds(2, Mb2 - 2)])
    hm2_ref[:, pl.ds(Mb2 - 2, 2)] = jnp.zeros((64, 2), hm2_ref.dtype)
    _pool_rows(o_ref, hm2_ref, 64, Pf, 22, 0)


def _conv34_kernel(x_ref, w3_ref, b3_ref, w4_ref, b4_ref, o_ref,
                   ps3_ref, x4_ref, ps4_ref, y4_ref, hm4_ref):
    Pf = 24
    Mb3 = 20 * Pf
    _fill_pstack(ps3_ref, x_ref, 64, Pf, Mb3, 0)
    z3 = jnp.dot(w3_ref[...], ps3_ref[...],
                 preferred_element_type=jnp.float32)
    y3 = jnp.maximum(z3 + b3_ref[...], 0.0)
    col = jax.lax.rem(jax.lax.broadcasted_iota(jnp.int32, (1, Mb3), 1), Pf)
    y3 = jnp.where(col < 20, y3, 0.0).astype(x4_ref.dtype)
    x4_ref[:, pl.ds(0, Pf + 1)] = jnp.zeros((128, Pf + 1), x4_ref.dtype)
    x4_ref[:, pl.ds(Pf + 1, Mb3)] = y3
    x4_ref[:, pl.ds(Pf + 1 + Mb3, 3 * Pf - 1)] = (
        jnp.zeros((128, 3 * Pf - 1), x4_ref.dtype))
    Mb4 = 18 * Pf
    for dy in range(KS):
        for dx in range(KS):
            t = dy * KS + dx
            ps4_ref[pl.ds(t * 128, 128), :] = (
                x4_ref[:, pl.ds(dy * Pf + dx, Mb4)])
    z4 = jnp.dot(w4_ref[...], ps4_ref[...],
                 preferred_element_type=jnp.float32)
    y4_ref[...] = jnp.maximum(z4 + b4_ref[...], 0.0).astype(y4_ref.dtype)
    hm4_ref[:, pl.ds(0, Mb4 - 2)] = jnp.maximum(
        jnp.maximum(y4_ref[:, pl.ds(0, Mb4 - 2)],
                    y4_ref[:, pl.ds(1, Mb4 - 2)]),
        y4_ref[:, pl.ds(2, Mb4 - 2)])
    hm4_ref[:, pl.ds(Mb4 - 2, 2)] = jnp.zeros((64, 2), hm4_ref.dtype)
    _pool_rows(o_ref, hm4_ref, 64, Pf, 6, 0)


def _linear_kernel(x_ref, w1_ref, b1_ref, w2_ref, b2_ref, w3_ref, b3_ref,
                   w4_ref, b4_ref, o_ref):
    h = jnp.maximum(
        jnp.dot(x_ref[...], w1_ref[...], preferred_element_type=jnp.float32)
        + b1_ref[...], 0.0).astype(jnp.bfloat16)
    h = jnp.maximum(
        jnp.dot(h, w2_ref[...], preferred_element_type=jnp.float32)
        + b2_ref[...], 0.0).astype(jnp.bfloat16)
    h = jnp.maximum(
        jnp.dot(h, w3_ref[...], preferred_element_type=jnp.float32)
        + b3_ref[...], 0.0).astype(jnp.bfloat16)
    o_ref[...] = (jnp.dot(h, w4_ref[...], preferred_element_type=jnp.float32)
                  + b4_ref[...])


def _fold_w(w, Cs):
    Cout, Cin = w.shape[0], w.shape[1]
    wf = jnp.pad(w, ((0, 0), (0, Cs - Cin), (0, 0), (0, 0)))
    wf = jnp.transpose(wf, (0, 2, 3, 1)).reshape(Cout, KS * KS * Cs)
    return wf.astype(jnp.bfloat16)


@jax.jit
def _forward(x, cw0, cb0, cw1, cb1, cw2, cb2, cw3, cb3, cw4, cb4,
             lw0, lb0, lw1, lb1, lw2, lb2, lw3, lb3):
    N = x.shape[0]
    cp = pltpu.CompilerParams(dimension_semantics=("parallel",),
                              vmem_limit_bytes=48 << 20)

    xp = jnp.pad(x, ((0, 0), (0, 5), (1, 2), (1, 1)))
    xp = xp.reshape(N, 8, 215 * 214).astype(jnp.bfloat16)
    y0 = pl.pallas_call(
        _conv0_kernel,
        out_shape=jax.ShapeDtypeStruct((N, 16, 70 * 214), jnp.bfloat16),
        grid=(N,),
        in_specs=[
            pl.BlockSpec((1, 8, 215 * 214), lambda n: (n, 0, 0)),
            pl.BlockSpec((16, 200), lambda n: (0, 0)),
            pl.BlockSpec((16, 1), lambda n: (0, 0)),
        ],
        out_specs=pl.BlockSpec((1, 16, 70 * 214), lambda n: (n, 0, 0)),
        scratch_shapes=[pltpu.VMEM((200, 105 * 214), jnp.bfloat16),
                        pltpu.VMEM((16, 105 * 214), jnp.bfloat16),
                        pltpu.VMEM((16, 105 * 214), jnp.bfloat16)],
        compiler_params=cp,
    )(xp, _fold_w(cw0, 8), cb0.reshape(16, 1).astype(jnp.float32))
    x1 = y0.reshape(N, 16, 70, 214)[:, :, :, 0:210:3]
    x1 = jnp.pad(x1, ((0, 0), (0, 0), (1, 3), (1, 1))).reshape(N, 16, 74 * 72)

    y2 = pl.pallas_call(
        _conv12_kernel,
        out_shape=jax.ShapeDtypeStruct((N, 64, 22 * 72), jnp.bfloat16),
        grid=(N,),
        in_specs=[
            pl.BlockSpec((1, 16, 74 * 72), lambda n: (n, 0, 0)),
            pl.BlockSpec((32, 400), lambda n: (0, 0)),
            pl.BlockSpec((32, 1), lambda n: (0, 0)),
            pl.BlockSpec((64, 800), lambda n: (0, 0)),
            pl.BlockSpec((64, 1), lambda n: (0, 0)),
        ],
        out_specs=pl.BlockSpec((1, 64, 22 * 72), lambda n: (n, 0, 0)),
        scratch_shapes=[pltpu.VMEM((400, 68 * 72), jnp.bfloat16),
                        pltpu.VMEM((32, 72 * 72), jnp.bfloat16),
                        pltpu.VMEM((800, 66 * 72), jnp.bfloat16),
                        pltpu.VMEM((64, 66 * 72), jnp.bfloat16),
                        pltpu.VMEM((64, 66 * 72), jnp.bfloat16)],
        compiler_params=cp,
    )(x1, _fold_w(cw1, 16), cb1.reshape(32, 1).astype(jnp.float32),
      _fold_w(cw2, 32), cb2.reshape(64, 1).astype(jnp.float32))
    x3 = y2.reshape(N, 64, 22, 72)[:, :, :, 0:66:3]
    x3 = jnp.pad(x3, ((0, 0), (0, 0), (1, 2), (1, 1))).reshape(N, 64, 25 * 24)

    y4 = pl.pallas_call(
        _conv34_kernel,
        out_shape=jax.ShapeDtypeStruct((N, 64, 6 * 24), jnp.bfloat16),
        grid=(N,),
        in_specs=[
            pl.BlockSpec((1, 64, 25 * 24), lambda n: (n, 0, 0)),
            pl.BlockSpec((128, 1600), lambda n: (0, 0)),
            pl.BlockSpec((128, 1), lambda n: (0, 0)),
            pl.BlockSpec((64, 3200), lambda n: (0, 0)),
            pl.BlockSpec((64, 1), lambda n: (0, 0)),
        ],
        out_specs=pl.BlockSpec((1, 64, 6 * 24), lambda n: (n, 0, 0)),
        scratch_shapes=[pltpu.VMEM((1600, 20 * 24), jnp.bfloat16),
                        pltpu.VMEM((128, 24 * 24), jnp.bfloat16),
                        pltpu.VMEM((3200, 18 * 24), jnp.bfloat16),
                        pltpu.VMEM((64, 18 * 24), jnp.bfloat16),
                        pltpu.VMEM((64, 18 * 24), jnp.bfloat16)],
        compiler_params=cp,
    )(x3, _fold_w(cw3, 64), cb3.reshape(128, 1).astype(jnp.float32),
      _fold_w(cw4, 128), cb4.reshape(64, 1).astype(jnp.float32))
    feat = y4.reshape(N, 64, 6, 24)[:, :, :, 0:18:3].reshape(N, 2304)

    HB = N // 2
    w1, w2 = lw0.astype(jnp.bfloat16), lw1.astype(jnp.bfloat16)
    w3 = jnp.pad(lw2, ((0, 0), (0, 8))).astype(jnp.bfloat16)
    w4 = jnp.pad(lw3, ((0, 8), (0, 6))).astype(jnp.bfloat16)
    b1 = lb0.reshape(1, -1).astype(jnp.float32)
    b2 = lb1.reshape(1, -1).astype(jnp.float32)
    b3 = jnp.pad(lb2, (0, 8)).reshape(1, -1).astype(jnp.float32)
    b4 = jnp.pad(lb3, (0, 6)).reshape(1, -1).astype(jnp.float32)
    out = pl.pallas_call(
        _linear_kernel,
        out_shape=jax.ShapeDtypeStruct((N, 8), jnp.float32),
        grid=(2,),
        in_specs=[
            pl.BlockSpec((HB, 2304), lambda i: (i, 0)),
            pl.BlockSpec((2304, 1500), lambda i: (0, 0)),
            pl.BlockSpec((1, 1500), lambda i: (0, 0)),
            pl.BlockSpec((1500, 600), lambda i: (0, 0)),
            pl.BlockSpec((1, 600), lambda i: (0, 0)),
            pl.BlockSpec((600, 48), lambda i: (0, 0)),
            pl.BlockSpec((1, 48), lambda i: (0, 0)),
            pl.BlockSpec((48, 8), lambda i: (0, 0)),
            pl.BlockSpec((1, 8), lambda i: (0, 0)),
        ],
        out_specs=pl.BlockSpec((HB, 8), lambda i: (i, 0)),
        compiler_params=pltpu.CompilerParams(
            dimension_semantics=("parallel",),
            vmem_limit_bytes=48 << 20),
    )(feat, w1, b1, w2, b2, w3, b3, w4, b4)
    return out[:, :2]


def kernel(x, cw0, cb0, cw1, cb1, cw2, cb2, cw3, cb3, cw4, cb4,
           lw0, lb0, lw1, lb1, lw2, lb2, lw3, lb3):
    return _forward(x, cw0, cb0, cw1, cb1, cw2, cb2, cw3, cb3, cw4, cb4,
                    lw0, lb0, lw1, lb1, lw2, lb2, lw3, lb3)
